# Initial kernel scaffold; baseline (speedup 1.0000x reference)
import jax
import jax.numpy as jnp
from jax.experimental import pallas as pl


def kernel(phy_x, phy_edge_index, fun_x, params):
    raise NotImplementedError("write your pallas kernel here")


# speedup vs baseline: 1.5955x; 1.5955x over previous
import functools

import jax
import jax.numpy as jnp
from jax import lax
from jax.experimental import pallas as pl
from jax.experimental.pallas import tpu as pltpu

_T = 10
_N = 50
_B = 8
_M = 500
_MP = 512
_TOTAL = 4000
_NP = 4096
_IN = 128
_HID = 256
_FD = 64
_H2 = 128
_NCLS = 8
_NE = 560000


def _add_body(x_ref, e_ref, o_ref):
    o_ref[...] = x_ref[...] + e_ref[...]


def _embed_add(x, emb):
    return pl.pallas_call(
        _add_body,
        out_shape=jax.ShapeDtypeStruct(x.shape, jnp.float32),
    )(x, emb)


def _phy_mm_body(a_ref, xk_ref, xi_ref, w0_ref, w1_ref, b_ref, out_ref,
                 acc_ref, *, nk):
    k = pl.program_id(1)

    @pl.when(k == 0)
    def _():
        acc_ref[...] = jnp.zeros_like(acc_ref)

    acc_ref[...] += jnp.dot(a_ref[...], xk_ref[...],
                            preferred_element_type=jnp.float32)

    @pl.when(k == nk - 1)
    def _():
        out_ref[...] = (
            jnp.dot(xi_ref[...], w0_ref[...], preferred_element_type=jnp.float32)
            + jnp.dot(acc_ref[...], w1_ref[...], preferred_element_type=jnp.float32)
            + b_ref[...])


def _phy_mm(a, x, w0, w1, b):
    cin, cout = w0.shape
    ni, nk = _NP // 512, _NP // 512
    return pl.pallas_call(
        functools.partial(_phy_mm_body, nk=nk),
        grid=(ni, nk),
        in_specs=[
            pl.BlockSpec((512, 512), lambda i, k: (i, k)),
            pl.BlockSpec((512, cin), lambda i, k: (k, 0)),
            pl.BlockSpec((512, cin), lambda i, k: (i, 0)),
            pl.BlockSpec((cin, cout), lambda i, k: (0, 0)),
            pl.BlockSpec((cin, cout), lambda i, k: (0, 0)),
            pl.BlockSpec((1, cout), lambda i, k: (0, 0)),
        ],
        out_specs=pl.BlockSpec((512, cout), lambda i, k: (i, 0)),
        out_shape=jax.ShapeDtypeStruct((_NP, cout), jnp.float32),
        scratch_shapes=[pltpu.VMEM((512, cin), jnp.float32)],
    )(a, x, x, w0, w1, b)


def _bnres_body(pre_ref, x_ref, wres_ref, bres_ref, g_ref, bt_ref, o_ref,
                *, fun_layout):
    pre = pre_ref[...]
    rows = lax.broadcasted_iota(jnp.int32, pre.shape, 0)
    if fun_layout:
        valid = (rows % _MP) < _M
    else:
        valid = rows < _TOTAL
    vf = valid.astype(jnp.float32)
    cnt = float(_TOTAL)
    s = jnp.sum(pre * vf, axis=0, keepdims=True)
    ss = jnp.sum(pre * pre * vf, axis=0, keepdims=True)
    mean = s / cnt
    var = ss / cnt - mean * mean
    bn = (pre - mean) * lax.rsqrt(var + 1e-5) * g_ref[...] + bt_ref[...]
    res = jnp.dot(x_ref[...], wres_ref[...],
                  preferred_element_type=jnp.float32) + bres_ref[...]
    o_ref[...] = jnp.maximum(bn + res, 0.0) * vf


def _bnres(pre, x, wres_t, bres, gamma, beta, fun_layout):
    cout = pre.shape[1]
    return pl.pallas_call(
        functools.partial(_bnres_body, fun_layout=fun_layout),
        out_shape=jax.ShapeDtypeStruct(pre.shape, jnp.float32),
    )(pre, x, wres_t, bres, gamma, beta)


def _fun_adj_body(x_ref, o_ref):
    x = x_ref[0]
    s = jnp.sum(x * x, axis=1, keepdims=True)
    n = jnp.maximum(jnp.sqrt(s), 1e-12)
    xn = x / n
    adj = lax.dot_general(xn, xn, (((1,), (1,)), ((), ())),
                          preferred_element_type=jnp.float32)
    ri = lax.broadcasted_iota(jnp.int32, (_MP, _MP), 0)
    ci = lax.broadcasted_iota(jnp.int32, (_MP, _MP), 1)
    diag = ri == ci
    idm = jnp.where(diag, 1.0, 0.0)
    adj = jnp.where(diag, 0.0, adj)

    neg = -1e30
    work = jnp.where(adj > 0.0, adj, neg)
    sel = jnp.zeros((_MP, _MP), jnp.float32)
    for _ in range(8):
        m = jnp.max(work, axis=1, keepdims=True)
        ismax = (work == m) & (m > 0.0)
        cif = ci.astype(jnp.float32)
        first = jnp.min(jnp.where(ismax, cif, 1e9), axis=1, keepdims=True)
        one = cif == first
        sel = jnp.where(one, 1.0, sel)
        work = jnp.where(one, neg, work)

    sel_t = lax.dot_general(sel, idm, (((0,), (0,)), ((), ())),
                            preferred_element_type=jnp.float32)
    adj_t = lax.dot_general(adj, idm, (((0,), (0,)), ((), ())),
                            preferred_element_type=jnp.float32)
    sparse = (adj * sel + adj_t * sel_t) * 0.5
    mask = (sparse > 0.0).astype(jnp.float32)
    cnt = jnp.sum(mask)
    eyep = jnp.where(diag & (ri < _M), 1.0, 0.0)
    flag = (cnt > 0.0).astype(jnp.float32)
    maskf = mask * flag + eyep * (1.0 - flag)

    deg = jnp.sum(maskf, axis=1, keepdims=True)
    dinv = jnp.where(deg > 0.0, lax.rsqrt(jnp.maximum(deg, 1e-12)), 0.0)
    dinv_r = lax.dot_general(dinv, idm, (((0,), (0,)), ((), ())),
                             preferred_element_type=jnp.float32)
    o_ref[0] = -(dinv * dinv_r) * maskf


def _fun_adj(x3):
    return pl.pallas_call(
        _fun_adj_body,
        grid=(_B,),
        in_specs=[pl.BlockSpec((1, _MP, _IN), lambda b: (b, 0, 0))],
        out_specs=pl.BlockSpec((1, _MP, _MP), lambda b: (b, 0, 0)),
        out_shape=jax.ShapeDtypeStruct((_B, _MP, _MP), jnp.float32),
    )(x3)


def _fun_mm_body(wn_ref, x_ref, w0_ref, w1_ref, b_ref, o_ref):
    x = x_ref[...]
    lx = jnp.dot(wn_ref[0], x, preferred_element_type=jnp.float32)
    o_ref[...] = (jnp.dot(x, w0_ref[...], preferred_element_type=jnp.float32)
                  + jnp.dot(lx, w1_ref[...], preferred_element_type=jnp.float32)
                  + b_ref[...])


def _fun_mm(wn, x, w0, w1, b):
    cin, cout = w0.shape
    return pl.pallas_call(
        _fun_mm_body,
        grid=(_B,),
        in_specs=[
            pl.BlockSpec((1, _MP, _MP), lambda bb: (bb, 0, 0)),
            pl.BlockSpec((_MP, cin), lambda bb: (bb, 0)),
            pl.BlockSpec((cin, cout), lambda bb: (0, 0)),
            pl.BlockSpec((cin, cout), lambda bb: (0, 0)),
            pl.BlockSpec((1, cout), lambda bb: (0, 0)),
        ],
        out_specs=pl.BlockSpec((_MP, cout), lambda bb: (bb, 0)),
        out_shape=jax.ShapeDtypeStruct((_B * _MP, cout), jnp.float32),
    )(wn, x, w0, w1, b)


def _tail_body(xp_ref, xf_ref, pool_ref, wsa1_ref, bsa1_ref, wsa2_ref,
               bsa2_ref, wth1_ref, bth1_ref, wth2_ref, bth2_ref, gate_ref,
               wtr_ref, btr_ref, wfc1_ref, bfc1_ref, wfc2_ref, bfc2_ref,
               logit_ref, fused_ref):
    xp = xp_ref[...]
    xf = xf_ref[...]
    sp = jnp.concatenate([xp, xf], axis=1)
    h = jnp.maximum(jnp.dot(sp, wsa1_ref[...],
                            preferred_element_type=jnp.float32)
                    + bsa1_ref[...], 0.0)
    attn = jax.nn.sigmoid(jnp.dot(h, wsa2_ref[...],
                                  preferred_element_type=jnp.float32)
                          + bsa2_ref[...])
    fsp = sp * attn

    rows = lax.broadcasted_iota(jnp.int32, (_TOTAL, 1), 0)
    rowt = (rows // _N) % _T
    h2 = jnp.zeros((_TOTAL, _FD), jnp.float32)
    for t in range(_T):
        ht = jnp.dot(fsp, wth1_ref[t], preferred_element_type=jnp.float32)
        h2 = h2 + jnp.where(rowt == t, ht + bth1_ref[t], 0.0)
    h2 = jnp.maximum(h2, 0.0)
    a2 = jnp.zeros((_TOTAL, 1), jnp.float32)
    for t in range(_T):
        at = jnp.sum(h2 * wth2_ref[t], axis=1, keepdims=True)
        a2 = a2 + jnp.where(rowt == t, at + bth2_ref[t], 0.0)
    alpha = jax.nn.sigmoid(a2)

    pool = pool_ref[...]
    g_phy = jnp.dot(pool, alpha * xp, preferred_element_type=jnp.float32)
    g_fun = jnp.dot(pool, (1.0 - alpha) * xf, preferred_element_type=jnp.float32)
    gate = gate_ref[...]
    comb = gate * g_phy + (1.0 - gate) * g_fun
    fused = jnp.maximum(jnp.dot(comb, wtr_ref[...],
                                preferred_element_type=jnp.float32)
                        + btr_ref[...], 0.0)
    hfc = jnp.maximum(jnp.dot(fused, wfc1_ref[...],
                              preferred_element_type=jnp.float32)
                      + bfc1_ref[...], 0.0)
    lg = jnp.dot(hfc, wfc2_ref[...],
                 preferred_element_type=jnp.float32) + bfc2_ref[...]
    cols = lax.broadcasted_iota(jnp.int32, (_B, _H2), 1)
    cmask = cols < _NCLS
    mx = jnp.max(jnp.where(cmask, lg, -1e30), axis=1, keepdims=True)
    e = jnp.where(cmask, jnp.exp(lg - mx), 0.0)
    se = jnp.sum(e, axis=1, keepdims=True)
    logit_ref[...] = lg - mx - jnp.log(se)
    fused_ref[...] = fused


def _tail(xp, xf, p):
    wth1_t = jnp.transpose(p["W_th1"], (0, 2, 1))
    bth1 = p["b_th1"][:, None, :]
    wth2 = p["W_th2"]
    bth2 = p["b_th2"][:, None, :]
    gate = jnp.clip(p["gate"], 0.0, 1.0)
    wfc2_t = jnp.zeros((_H2, _H2), jnp.float32).at[:, :_NCLS].set(p["W_fc2"].T)
    bfc2 = jnp.zeros((1, _H2), jnp.float32).at[0, :_NCLS].set(p["b_fc2"])
    bidx = jnp.arange(_TOTAL, dtype=jnp.int32) // _M
    pool = (bidx[None, :] == jnp.arange(_B, dtype=jnp.int32)[:, None]
            ).astype(jnp.float32) * (1.0 / _M)
    logits_pad, fused = pl.pallas_call(
        _tail_body,
        out_shape=(jax.ShapeDtypeStruct((_B, _H2), jnp.float32),
                   jax.ShapeDtypeStruct((_B, _H2), jnp.float32)),
    )(xp, xf, pool,
      p["W_sa1"].T, p["b_sa1"][None, :], p["W_sa2"].T, p["b_sa2"][None, :],
      wth1_t, bth1, wth2, bth2, gate.reshape(1, 1),
      p["W_tr"].T, p["b_tr"][None, :], p["W_fc1"].T, p["b_fc1"][None, :],
      wfc2_t, bfc2)
    return logits_pad[:, :_NCLS], gate, fused


def _build_adj(src, dst):
    ones = jnp.ones((_NE,), jnp.float32)
    deg = jnp.zeros((_TOTAL,), jnp.float32).at[src].add(ones)
    dinv = jnp.where(deg > 0, 1.0 / jnp.sqrt(jnp.maximum(deg, 1e-12)), 0.0)
    w = -(dinv[src] * dinv[dst])
    a = jnp.zeros((_NP, _NP), jnp.float32).at[dst, src].add(w)
    return a


def kernel(phy_x, phy_edge_index, fun_x, params):
    p = params
    src = phy_edge_index[0]
    dst = phy_edge_index[1]
    a = _build_adj(src, dst)

    emb = (p["temp_embed"] + p["spat_embed"]).reshape(_M, _IN)
    emb_phy = jnp.broadcast_to(emb[None], (_B, _M, _IN)).reshape(_TOTAL, _IN)
    emb_phy = jnp.pad(emb_phy, ((0, _NP - _TOTAL), (0, 0)))
    emb_fun = jnp.pad(emb[None], ((0, 0), (0, _MP - _M), (0, 0)))
    emb_fun = jnp.broadcast_to(emb_fun, (_B, _MP, _IN)).reshape(_B * _MP, _IN)

    xp = jnp.pad(phy_x, ((0, _NP - _TOTAL), (0, 0)))
    xf3 = jnp.pad(fun_x.reshape(_B, _M, _IN), ((0, 0), (0, _MP - _M), (0, 0)))
    xf = xf3.reshape(_B * _MP, _IN)

    x = _embed_add(xp, emb_phy)
    y = _embed_add(xf, emb_fun)

    wn = _fun_adj(xf3)

    def pad_w(wmat, cin_p, cout_p):
        w2 = jnp.zeros((cin_p, cout_p), jnp.float32)
        return w2.at[:wmat.shape[0], :wmat.shape[1]].set(wmat)

    for li, lp in enumerate(p["phy_convs"]):
        cin, cout = lp["W0"].shape
        cout_p = max(cout, 128)
        w0 = pad_w(lp["W0"], cin, cout_p)
        w1 = pad_w(lp["W1"], cin, cout_p)
        b = pad_w(lp["b"][None, :], 1, cout_p)
        wres_t = pad_w(lp["Wres"].T, cin, cout_p)
        bres = pad_w(lp["bres"][None, :], 1, cout_p)
        gamma = pad_w(lp["gamma"][None, :], 1, cout_p)
        beta = pad_w(lp["beta"][None, :], 1, cout_p)
        pre = _phy_mm(a, x, w0, w1, b)
        x = _bnres(pre, x, wres_t, bres, gamma, beta, fun_layout=False)

    for li, lp in enumerate(p["fun_convs"]):
        cin, cout = lp["W0"].shape
        cout_p = max(cout, 128)
        w0 = pad_w(lp["W0"], cin, cout_p)
        w1 = pad_w(lp["W1"], cin, cout_p)
        b = pad_w(lp["b"][None, :], 1, cout_p)
        wres_t = pad_w(lp["Wres"].T, cin, cout_p)
        bres = pad_w(lp["bres"][None, :], 1, cout_p)
        gamma = pad_w(lp["gamma"][None, :], 1, cout_p)
        beta = pad_w(lp["beta"][None, :], 1, cout_p)
        pre = _fun_mm(wn, y, w0, w1, b)
        y = _bnres(pre, y, wres_t, bres, gamma, beta, fun_layout=True)

    x_phy = x[:_TOTAL, :_FD]
    x_fun = y.reshape(_B, _MP, -1)[:, :_M, :_FD].reshape(_TOTAL, _FD)

    return _tail(x_phy, x_fun, p)


# speedup vs baseline: 3.1468x; 1.9723x over previous
import functools

import jax
import jax.numpy as jnp
from jax import lax
from jax.experimental import pallas as pl
from jax.experimental.pallas import tpu as pltpu
from jax.experimental.pallas import tpu_sc as plsc

_T = 10
_N = 50
_B = 8
_M = 500
_MP = 512
_TOTAL = 4000
_NP = 4096
_IN = 128
_HID = 256
_FD = 64
_H2 = 128
_NCLS = 8
_NE = 560000


def _add_body(x_ref, e_ref, o_ref):
    o_ref[...] = x_ref[...] + e_ref[...]


def _embed_add(x, emb):
    return pl.pallas_call(
        _add_body,
        out_shape=jax.ShapeDtypeStruct(x.shape, jnp.float32),
    )(x, emb)


def _phy_mm_body(a_ref, xk_ref, xi_ref, dk_ref, di_ref, w0_ref, w1_ref,
                 b_ref, out_ref, acc_ref, *, nk):
    k = pl.program_id(1)

    @pl.when(k == 0)
    def _():
        acc_ref[...] = jnp.zeros_like(acc_ref)

    acc_ref[...] += jnp.dot(a_ref[...], xk_ref[...] * dk_ref[...],
                            preferred_element_type=jnp.float32)

    @pl.when(k == nk - 1)
    def _():
        out_ref[...] = (
            jnp.dot(xi_ref[...], w0_ref[...], preferred_element_type=jnp.float32)
            + jnp.dot(acc_ref[...] * (-di_ref[...]), w1_ref[...],
                      preferred_element_type=jnp.float32)
            + b_ref[...])


def _phy_mm(a, dinv_col, x, w0, w1, b):
    cin, cout = w0.shape
    ni, nk = _NP // 512, _NP // 512
    return pl.pallas_call(
        functools.partial(_phy_mm_body, nk=nk),
        grid=(ni, nk),
        in_specs=[
            pl.BlockSpec((512, 512), lambda i, k: (i, k)),
            pl.BlockSpec((512, cin), lambda i, k: (k, 0)),
            pl.BlockSpec((512, cin), lambda i, k: (i, 0)),
            pl.BlockSpec((512, 1), lambda i, k: (k, 0)),
            pl.BlockSpec((512, 1), lambda i, k: (i, 0)),
            pl.BlockSpec((cin, cout), lambda i, k: (0, 0)),
            pl.BlockSpec((cin, cout), lambda i, k: (0, 0)),
            pl.BlockSpec((1, cout), lambda i, k: (0, 0)),
        ],
        out_specs=pl.BlockSpec((512, cout), lambda i, k: (i, 0)),
        out_shape=jax.ShapeDtypeStruct((_NP, cout), jnp.float32),
        scratch_shapes=[pltpu.VMEM((512, cin), jnp.float32)],
    )(a, x, x, dinv_col, dinv_col, w0, w1, b)


def _bnres_body(pre_ref, x_ref, wres_ref, bres_ref, g_ref, bt_ref, o_ref,
                *, fun_layout):
    pre = pre_ref[...]
    rows = lax.broadcasted_iota(jnp.int32, pre.shape, 0)
    if fun_layout:
        valid = (rows % _MP) < _M
    else:
        valid = rows < _TOTAL
    vf = valid.astype(jnp.float32)
    cnt = float(_TOTAL)
    s = jnp.sum(pre * vf, axis=0, keepdims=True)
    ss = jnp.sum(pre * pre * vf, axis=0, keepdims=True)
    mean = s / cnt
    var = ss / cnt - mean * mean
    bn = (pre - mean) * lax.rsqrt(var + 1e-5) * g_ref[...] + bt_ref[...]
    res = jnp.dot(x_ref[...], wres_ref[...],
                  preferred_element_type=jnp.float32) + bres_ref[...]
    o_ref[...] = jnp.maximum(bn + res, 0.0) * vf


def _bnres(pre, x, wres_t, bres, gamma, beta, fun_layout):
    cout = pre.shape[1]
    return pl.pallas_call(
        functools.partial(_bnres_body, fun_layout=fun_layout),
        out_shape=jax.ShapeDtypeStruct(pre.shape, jnp.float32),
    )(pre, x, wres_t, bres, gamma, beta)


def _fun_adj_body(x_ref, o_ref):
    x = x_ref[0]
    s = jnp.sum(x * x, axis=1, keepdims=True)
    n = jnp.maximum(jnp.sqrt(s), 1e-12)
    xn = x / n
    adj = lax.dot_general(xn, xn, (((1,), (1,)), ((), ())),
                          preferred_element_type=jnp.float32)
    ri = lax.broadcasted_iota(jnp.int32, (_MP, _MP), 0)
    ci = lax.broadcasted_iota(jnp.int32, (_MP, _MP), 1)
    diag = ri == ci
    idm = jnp.where(diag, 1.0, 0.0)
    adj = jnp.where(diag, 0.0, adj)

    neg = -1e30
    work = jnp.where(adj > 0.0, adj, neg)
    sel = jnp.zeros((_MP, _MP), jnp.float32)
    for _ in range(8):
        m = jnp.max(work, axis=1, keepdims=True)
        ismax = (work == m) & (m > 0.0)
        cif = ci.astype(jnp.float32)
        first = jnp.min(jnp.where(ismax, cif, 1e9), axis=1, keepdims=True)
        one = cif == first
        sel = jnp.where(one, 1.0, sel)
        work = jnp.where(one, neg, work)

    sel_t = lax.dot_general(sel, idm, (((0,), (0,)), ((), ())),
                            preferred_element_type=jnp.float32)
    adj_t = lax.dot_general(adj, idm, (((0,), (0,)), ((), ())),
                            preferred_element_type=jnp.float32)
    sparse = (adj * sel + adj_t * sel_t) * 0.5
    mask = (sparse > 0.0).astype(jnp.float32)
    cnt = jnp.sum(mask)
    eyep = jnp.where(diag & (ri < _M), 1.0, 0.0)
    flag = (cnt > 0.0).astype(jnp.float32)
    maskf = mask * flag + eyep * (1.0 - flag)

    deg = jnp.sum(maskf, axis=1, keepdims=True)
    dinv = jnp.where(deg > 0.0, lax.rsqrt(jnp.maximum(deg, 1e-12)), 0.0)
    dinv_r = lax.dot_general(dinv, idm, (((0,), (0,)), ((), ())),
                             preferred_element_type=jnp.float32)
    o_ref[0] = -(dinv * dinv_r) * maskf


def _fun_adj(x3):
    return pl.pallas_call(
        _fun_adj_body,
        grid=(_B,),
        in_specs=[pl.BlockSpec((1, _MP, _IN), lambda b: (b, 0, 0))],
        out_specs=pl.BlockSpec((1, _MP, _MP), lambda b: (b, 0, 0)),
        out_shape=jax.ShapeDtypeStruct((_B, _MP, _MP), jnp.float32),
    )(x3)


def _fun_mm_body(wn_ref, x_ref, w0_ref, w1_ref, b_ref, o_ref):
    x = x_ref[...]
    lx = jnp.dot(wn_ref[0], x, preferred_element_type=jnp.float32)
    o_ref[...] = (jnp.dot(x, w0_ref[...], preferred_element_type=jnp.float32)
                  + jnp.dot(lx, w1_ref[...], preferred_element_type=jnp.float32)
                  + b_ref[...])


def _fun_mm(wn, x, w0, w1, b):
    cin, cout = w0.shape
    return pl.pallas_call(
        _fun_mm_body,
        grid=(_B,),
        in_specs=[
            pl.BlockSpec((1, _MP, _MP), lambda bb: (bb, 0, 0)),
            pl.BlockSpec((_MP, cin), lambda bb: (bb, 0)),
            pl.BlockSpec((cin, cout), lambda bb: (0, 0)),
            pl.BlockSpec((cin, cout), lambda bb: (0, 0)),
            pl.BlockSpec((1, cout), lambda bb: (0, 0)),
        ],
        out_specs=pl.BlockSpec((_MP, cout), lambda bb: (bb, 0)),
        out_shape=jax.ShapeDtypeStruct((_B * _MP, cout), jnp.float32),
    )(wn, x, w0, w1, b)


def _tail_body(xp_ref, xf_ref, pool_ref, wsa1_ref, bsa1_ref, wsa2_ref,
               bsa2_ref, wth1_ref, bth1_ref, wth2_ref, bth2_ref, gate_ref,
               wtr_ref, btr_ref, wfc1_ref, bfc1_ref, wfc2_ref, bfc2_ref,
               logit_ref, fused_ref):
    xp = xp_ref[...]
    xf = xf_ref[...]
    sp = jnp.concatenate([xp, xf], axis=1)
    h = jnp.maximum(jnp.dot(sp, wsa1_ref[...],
                            preferred_element_type=jnp.float32)
                    + bsa1_ref[...], 0.0)
    attn = jax.nn.sigmoid(jnp.dot(h, wsa2_ref[...],
                                  preferred_element_type=jnp.float32)
                          + bsa2_ref[...])
    fsp = sp * attn

    rows = lax.broadcasted_iota(jnp.int32, (_TOTAL, 1), 0)
    rowt = (rows // _N) % _T
    h2 = jnp.zeros((_TOTAL, _FD), jnp.float32)
    for t in range(_T):
        ht = jnp.dot(fsp, wth1_ref[t], preferred_element_type=jnp.float32)
        h2 = h2 + jnp.where(rowt == t, ht + bth1_ref[t], 0.0)
    h2 = jnp.maximum(h2, 0.0)
    a2 = jnp.zeros((_TOTAL, 1), jnp.float32)
    for t in range(_T):
        at = jnp.sum(h2 * wth2_ref[t], axis=1, keepdims=True)
        a2 = a2 + jnp.where(rowt == t, at + bth2_ref[t], 0.0)
    alpha = jax.nn.sigmoid(a2)

    pool = pool_ref[...]
    g_phy = jnp.dot(pool, alpha * xp, preferred_element_type=jnp.float32)
    g_fun = jnp.dot(pool, (1.0 - alpha) * xf, preferred_element_type=jnp.float32)
    gate = gate_ref[...]
    comb = gate * g_phy + (1.0 - gate) * g_fun
    fused = jnp.maximum(jnp.dot(comb, wtr_ref[...],
                                preferred_element_type=jnp.float32)
                        + btr_ref[...], 0.0)
    hfc = jnp.maximum(jnp.dot(fused, wfc1_ref[...],
                              preferred_element_type=jnp.float32)
                      + bfc1_ref[...], 0.0)
    lg = jnp.dot(hfc, wfc2_ref[...],
                 preferred_element_type=jnp.float32) + bfc2_ref[...]
    cols = lax.broadcasted_iota(jnp.int32, (_B, _H2), 1)
    cmask = cols < _NCLS
    mx = jnp.max(jnp.where(cmask, lg, -1e30), axis=1, keepdims=True)
    e = jnp.where(cmask, jnp.exp(lg - mx), 0.0)
    se = jnp.sum(e, axis=1, keepdims=True)
    logit_ref[...] = lg - mx - jnp.log(se)
    fused_ref[...] = fused


def _tail(xp, xf, p):
    wth1_t = jnp.transpose(p["W_th1"], (0, 2, 1))
    bth1 = p["b_th1"][:, None, :]
    wth2 = p["W_th2"]
    bth2 = p["b_th2"][:, None, :]
    gate = jnp.clip(p["gate"], 0.0, 1.0)
    wfc2_t = jnp.zeros((_H2, _H2), jnp.float32).at[:, :_NCLS].set(p["W_fc2"].T)
    bfc2 = jnp.zeros((1, _H2), jnp.float32).at[0, :_NCLS].set(p["b_fc2"])
    bidx = jnp.arange(_TOTAL, dtype=jnp.int32) // _M
    pool = (bidx[None, :] == jnp.arange(_B, dtype=jnp.int32)[:, None]
            ).astype(jnp.float32) * (1.0 / _M)
    logits_pad, fused = pl.pallas_call(
        _tail_body,
        out_shape=(jax.ShapeDtypeStruct((_B, _H2), jnp.float32),
                   jax.ShapeDtypeStruct((_B, _H2), jnp.float32)),
    )(xp, xf, pool,
      p["W_sa1"].T, p["b_sa1"][None, :], p["W_sa2"].T, p["b_sa2"][None, :],
      wth1_t, bth1, wth2, bth2, gate.reshape(1, 1),
      p["W_tr"].T, p["b_tr"][None, :], p["W_fc1"].T, p["b_fc1"][None, :],
      wfc2_t, bfc2)
    return logits_pad[:, :_NCLS], gate, fused


_NE_PAD = 573440
_EPT = _NE_PAD // 32
_EPT_A = _NE_PAD // 16
_RB = 256
_NBLK = _NP // _RB
_BLK_ELEMS = _RB * _NP


def _sc_deg_body(src_hbm, zeros_hbm, out_hbm, src_v, vals_v, idx_v, deg_sh):
    c = lax.axis_index("c")
    t = lax.axis_index("s")
    base = (t * 2 + c) * _EPT
    pltpu.sync_copy(src_hbm.at[pl.ds(base, _EPT)], src_v)

    seg = _NP // 16
    pltpu.sync_copy(zeros_hbm.at[pl.ds(t * seg, seg)],
                    deg_sh.at[pl.ds(t * seg, seg)])
    ones16 = jnp.ones((16,), jnp.float32)
    for g in range(8):
        vals_v[pl.ds(g * 16, 16)] = ones16
    plsc.subcore_barrier()

    def chunk_it(cc, carry):
        for g in range(8):
            idx_v[pl.ds(g * 16, 16)] = src_v[pl.ds(cc * 128 + g * 16, 16)]
        pltpu.sync_copy(vals_v, deg_sh.at[idx_v], add=True)
        return carry
    lax.fori_loop(0, _EPT // 128, chunk_it, 0)
    plsc.subcore_barrier()

    pltpu.sync_copy(deg_sh.at[pl.ds(t * seg, seg)],
                    out_hbm.at[pl.ds(c * _NP + t * seg, seg)])


def _sc_deg(src_pad):
    mesh = plsc.VectorSubcoreMesh(core_axis_name="c", subcore_axis_name="s")
    zeros = jnp.zeros((_NP,), jnp.float32)
    f = functools.partial(
        pl.kernel, mesh=mesh,
        out_type=jax.ShapeDtypeStruct((2 * _NP,), jnp.float32),
        scratch_types=[
            pltpu.VMEM((_EPT,), jnp.int32),
            pltpu.VMEM((128,), jnp.float32),
            pltpu.VMEM((128,), jnp.int32),
            pltpu.VMEM_SHARED((_NP,), jnp.float32),
        ],
    )(_sc_deg_body)
    return f(src_pad, zeros).reshape(2, _NP)


def _dinv_body(degp_ref, o_ref):
    deg = jnp.sum(degp_ref[...], axis=0, keepdims=True)
    o_ref[...] = jnp.where(deg > 0.0,
                           lax.rsqrt(jnp.maximum(deg, 1e-12)), 0.0)


def _dinv_tc(deg_part):
    return pl.pallas_call(
        _dinv_body,
        out_shape=jax.ShapeDtypeStruct((1, _NP), jnp.float32),
    )(deg_part)


_ECH = 2560


def _sc_adj_body(src_hbm, dst_hbm, zeros_hbm, a_hbm,
                 src_v, dst_v, vals_v, idx_v, blk_sh):
    c = lax.axis_index("c")
    t = lax.axis_index("s")
    base = t * _EPT_A

    seg = _BLK_ELEMS // 16

    for pp in range(_NBLK // 2):
        row0 = c * (_NBLK // 2) * _RB + pp * _RB

        pltpu.sync_copy(zeros_hbm.at[pl.ds(t * seg, seg)],
                        blk_sh.at[pl.ds(t * seg, seg)])
        plsc.subcore_barrier()

        def chunk_it(cc, carry):
            e0 = base + cc * _ECH
            pltpu.sync_copy(src_hbm.at[pl.ds(e0, _ECH)], src_v)
            pltpu.sync_copy(dst_hbm.at[pl.ds(e0, _ECH)], dst_v)

            def sub_it(ss, carry2):
                for g in range(8):
                    off = ss * 128 + g * 16
                    s16 = src_v[pl.ds(off, 16)]
                    d16 = dst_v[pl.ds(off, 16)]
                    rel = d16 - row0
                    inr = (rel >= 0) & (rel < _RB)
                    flat = rel * _NP + s16
                    idx_v[pl.ds(g * 16, 16)] = jnp.where(inr, flat, 0)
                    vals_v[pl.ds(g * 16, 16)] = jnp.where(inr, 1.0, 0.0)
                pltpu.sync_copy(vals_v, blk_sh.at[idx_v], add=True)
                return carry2
            lax.fori_loop(0, _ECH // 128, sub_it, 0)
            return carry
        lax.fori_loop(0, _EPT_A // _ECH, chunk_it, 0)
        plsc.subcore_barrier()

        pltpu.sync_copy(blk_sh.at[pl.ds(t * seg, seg)],
                        a_hbm.at[pl.ds(row0 * _NP + t * seg, seg)])
        plsc.subcore_barrier()


def _sc_adj(src_pad, dst_pad):
    mesh = plsc.VectorSubcoreMesh(core_axis_name="c", subcore_axis_name="s")
    zeros = jnp.zeros((_BLK_ELEMS,), jnp.float32)
    f = functools.partial(
        pl.kernel, mesh=mesh,
        out_type=jax.ShapeDtypeStruct((_NP * _NP,), jnp.float32),
        scratch_types=[
            pltpu.VMEM((_ECH,), jnp.int32),
            pltpu.VMEM((_ECH,), jnp.int32),
            pltpu.VMEM((128,), jnp.float32),
            pltpu.VMEM((128,), jnp.int32),
            pltpu.VMEM_SHARED((_BLK_ELEMS,), jnp.float32),
        ],
    )(_sc_adj_body)
    return f(src_pad, dst_pad, zeros).reshape(_NP, _NP)


def _build_adj(src, dst):
    pad = jnp.full((_NE_PAD - _NE,), _NP - 1, jnp.int32)
    src_pad = jnp.concatenate([src, pad])
    dst_pad = jnp.concatenate([dst, pad])
    deg_part = _sc_deg(src_pad)
    dinv = _dinv_tc(deg_part).reshape(_NP, 1)
    cmat = _sc_adj(src_pad, dst_pad)
    return cmat, dinv


def kernel(phy_x, phy_edge_index, fun_x, params):
    p = params
    src = phy_edge_index[0]
    dst = phy_edge_index[1]
    a, dinv_col = _build_adj(src, dst)

    emb = (p["temp_embed"] + p["spat_embed"]).reshape(_M, _IN)
    emb_phy = jnp.broadcast_to(emb[None], (_B, _M, _IN)).reshape(_TOTAL, _IN)
    emb_phy = jnp.pad(emb_phy, ((0, _NP - _TOTAL), (0, 0)))
    emb_fun = jnp.pad(emb[None], ((0, 0), (0, _MP - _M), (0, 0)))
    emb_fun = jnp.broadcast_to(emb_fun, (_B, _MP, _IN)).reshape(_B * _MP, _IN)

    xp = jnp.pad(phy_x, ((0, _NP - _TOTAL), (0, 0)))
    xf3 = jnp.pad(fun_x.reshape(_B, _M, _IN), ((0, 0), (0, _MP - _M), (0, 0)))
    xf = xf3.reshape(_B * _MP, _IN)

    x = _embed_add(xp, emb_phy)
    y = _embed_add(xf, emb_fun)

    wn = _fun_adj(xf3)

    def pad_w(wmat, cin_p, cout_p):
        w2 = jnp.zeros((cin_p, cout_p), jnp.float32)
        return w2.at[:wmat.shape[0], :wmat.shape[1]].set(wmat)

    for li, lp in enumerate(p["phy_convs"]):
        cin, cout = lp["W0"].shape
        cout_p = max(cout, 128)
        w0 = pad_w(lp["W0"], cin, cout_p)
        w1 = pad_w(lp["W1"], cin, cout_p)
        b = pad_w(lp["b"][None, :], 1, cout_p)
        wres_t = pad_w(lp["Wres"].T, cin, cout_p)
        bres = pad_w(lp["bres"][None, :], 1, cout_p)
        gamma = pad_w(lp["gamma"][None, :], 1, cout_p)
        beta = pad_w(lp["beta"][None, :], 1, cout_p)
        pre = _phy_mm(a, dinv_col, x, w0, w1, b)
        x = _bnres(pre, x, wres_t, bres, gamma, beta, fun_layout=False)

    for li, lp in enumerate(p["fun_convs"]):
        cin, cout = lp["W0"].shape
        cout_p = max(cout, 128)
        w0 = pad_w(lp["W0"], cin, cout_p)
        w1 = pad_w(lp["W1"], cin, cout_p)
        b = pad_w(lp["b"][None, :], 1, cout_p)
        wres_t = pad_w(lp["Wres"].T, cin, cout_p)
        bres = pad_w(lp["bres"][None, :], 1, cout_p)
        gamma = pad_w(lp["gamma"][None, :], 1, cout_p)
        beta = pad_w(lp["beta"][None, :], 1, cout_p)
        pre = _fun_mm(wn, y, w0, w1, b)
        y = _bnres(pre, y, wres_t, bres, gamma, beta, fun_layout=True)

    x_phy = x[:_TOTAL, :_FD]
    x_fun = y.reshape(_B, _MP, -1)[:, :_M, :_FD].reshape(_TOTAL, _FD)

    return _tail(x_phy, x_fun, p)


# speedup vs baseline: 25.6348x; 8.1464x over previous
import functools

import jax
import jax.numpy as jnp
from jax import lax
from jax.experimental import pallas as pl
from jax.experimental.pallas import tpu as pltpu
from jax.experimental.pallas import tpu_sc as plsc

_T = 10
_N = 50
_B = 8
_M = 500
_MP = 512
_TOTAL = 4000
_NP = 4096
_IN = 128
_HID = 256
_FD = 64
_H2 = 128
_NCLS = 8
_NE = 560000


def _add_body(x_ref, e_ref, o_ref):
    o_ref[...] = x_ref[...] + e_ref[...]


def _embed_add(x, emb):
    return pl.pallas_call(
        _add_body,
        out_shape=jax.ShapeDtypeStruct(x.shape, jnp.float32),
    )(x, emb)


def _phy_mm_body(a_ref, xk_ref, xi_ref, dk_ref, di_ref, w0_ref, w1_ref,
                 b_ref, out_ref, acc_ref, *, nk):
    k = pl.program_id(1)

    @pl.when(k == 0)
    def _():
        acc_ref[...] = jnp.zeros_like(acc_ref)

    acc_ref[...] += jnp.dot(a_ref[...], xk_ref[...] * dk_ref[...],
                            preferred_element_type=jnp.float32)

    @pl.when(k == nk - 1)
    def _():
        out_ref[...] = (
            jnp.dot(xi_ref[...], w0_ref[...], preferred_element_type=jnp.float32)
            + jnp.dot(acc_ref[...] * (-di_ref[...]), w1_ref[...],
                      preferred_element_type=jnp.float32)
            + b_ref[...])


def _phy_mm(a, dinv_col, x, w0, w1, b):
    cin, cout = w0.shape
    ni, nk = _NP // 512, _NP // 512
    return pl.pallas_call(
        functools.partial(_phy_mm_body, nk=nk),
        grid=(ni, nk),
        in_specs=[
            pl.BlockSpec((512, 512), lambda i, k: (i, k)),
            pl.BlockSpec((512, cin), lambda i, k: (k, 0)),
            pl.BlockSpec((512, cin), lambda i, k: (i, 0)),
            pl.BlockSpec((512, 1), lambda i, k: (k, 0)),
            pl.BlockSpec((512, 1), lambda i, k: (i, 0)),
            pl.BlockSpec((cin, cout), lambda i, k: (0, 0)),
            pl.BlockSpec((cin, cout), lambda i, k: (0, 0)),
            pl.BlockSpec((1, cout), lambda i, k: (0, 0)),
        ],
        out_specs=pl.BlockSpec((512, cout), lambda i, k: (i, 0)),
        out_shape=jax.ShapeDtypeStruct((_NP, cout), jnp.float32),
        scratch_shapes=[pltpu.VMEM((512, cin), jnp.float32)],
    )(a, x, x, dinv_col, dinv_col, w0, w1, b)


def _bnres_body(pre_ref, x_ref, wres_ref, bres_ref, g_ref, bt_ref, o_ref,
                *, fun_layout):
    pre = pre_ref[...]
    rows = lax.broadcasted_iota(jnp.int32, pre.shape, 0)
    if fun_layout:
        valid = (rows % _MP) < _M
    else:
        valid = rows < _TOTAL
    vf = valid.astype(jnp.float32)
    cnt = float(_TOTAL)
    s = jnp.sum(pre * vf, axis=0, keepdims=True)
    ss = jnp.sum(pre * pre * vf, axis=0, keepdims=True)
    mean = s / cnt
    var = ss / cnt - mean * mean
    bn = (pre - mean) * lax.rsqrt(var + 1e-5) * g_ref[...] + bt_ref[...]
    res = jnp.dot(x_ref[...], wres_ref[...],
                  preferred_element_type=jnp.float32) + bres_ref[...]
    o_ref[...] = jnp.maximum(bn + res, 0.0) * vf


def _bnres(pre, x, wres_t, bres, gamma, beta, fun_layout):
    cout = pre.shape[1]
    return pl.pallas_call(
        functools.partial(_bnres_body, fun_layout=fun_layout),
        out_shape=jax.ShapeDtypeStruct(pre.shape, jnp.float32),
    )(pre, x, wres_t, bres, gamma, beta)


def _fun_adj_body(x_ref, o_ref):
    x = x_ref[0]
    s = jnp.sum(x * x, axis=1, keepdims=True)
    n = jnp.maximum(jnp.sqrt(s), 1e-12)
    xn = x / n
    adj = lax.dot_general(xn, xn, (((1,), (1,)), ((), ())),
                          preferred_element_type=jnp.float32)
    ri = lax.broadcasted_iota(jnp.int32, (_MP, _MP), 0)
    ci = lax.broadcasted_iota(jnp.int32, (_MP, _MP), 1)
    diag = ri == ci
    idm = jnp.where(diag, 1.0, 0.0)
    adj = jnp.where(diag, 0.0, adj)

    neg = -1e30
    work = jnp.where(adj > 0.0, adj, neg)
    sel = jnp.zeros((_MP, _MP), jnp.float32)
    for _ in range(8):
        m = jnp.max(work, axis=1, keepdims=True)
        ismax = (work == m) & (m > 0.0)
        cif = ci.astype(jnp.float32)
        first = jnp.min(jnp.where(ismax, cif, 1e9), axis=1, keepdims=True)
        one = cif == first
        sel = jnp.where(one, 1.0, sel)
        work = jnp.where(one, neg, work)

    sel_t = lax.dot_general(sel, idm, (((0,), (0,)), ((), ())),
                            preferred_element_type=jnp.float32)
    adj_t = lax.dot_general(adj, idm, (((0,), (0,)), ((), ())),
                            preferred_element_type=jnp.float32)
    sparse = (adj * sel + adj_t * sel_t) * 0.5
    mask = (sparse > 0.0).astype(jnp.float32)
    cnt = jnp.sum(mask)
    eyep = jnp.where(diag & (ri < _M), 1.0, 0.0)
    flag = (cnt > 0.0).astype(jnp.float32)
    maskf = mask * flag + eyep * (1.0 - flag)

    deg = jnp.sum(maskf, axis=1, keepdims=True)
    dinv = jnp.where(deg > 0.0, lax.rsqrt(jnp.maximum(deg, 1e-12)), 0.0)
    dinv_r = lax.dot_general(dinv, idm, (((0,), (0,)), ((), ())),
                             preferred_element_type=jnp.float32)
    o_ref[0] = -(dinv * dinv_r) * maskf


def _fun_adj(x3):
    return pl.pallas_call(
        _fun_adj_body,
        grid=(_B,),
        in_specs=[pl.BlockSpec((1, _MP, _IN), lambda b: (b, 0, 0))],
        out_specs=pl.BlockSpec((1, _MP, _MP), lambda b: (b, 0, 0)),
        out_shape=jax.ShapeDtypeStruct((_B, _MP, _MP), jnp.float32),
    )(x3)


def _fun_mm_body(wn_ref, x_ref, w0_ref, w1_ref, b_ref, o_ref):
    x = x_ref[...]
    lx = jnp.dot(wn_ref[0], x, preferred_element_type=jnp.float32)
    o_ref[...] = (jnp.dot(x, w0_ref[...], preferred_element_type=jnp.float32)
                  + jnp.dot(lx, w1_ref[...], preferred_element_type=jnp.float32)
                  + b_ref[...])


def _fun_mm(wn, x, w0, w1, b):
    cin, cout = w0.shape
    return pl.pallas_call(
        _fun_mm_body,
        grid=(_B,),
        in_specs=[
            pl.BlockSpec((1, _MP, _MP), lambda bb: (bb, 0, 0)),
            pl.BlockSpec((_MP, cin), lambda bb: (bb, 0)),
            pl.BlockSpec((cin, cout), lambda bb: (0, 0)),
            pl.BlockSpec((cin, cout), lambda bb: (0, 0)),
            pl.BlockSpec((1, cout), lambda bb: (0, 0)),
        ],
        out_specs=pl.BlockSpec((_MP, cout), lambda bb: (bb, 0)),
        out_shape=jax.ShapeDtypeStruct((_B * _MP, cout), jnp.float32),
    )(wn, x, w0, w1, b)


def _tail_body(xp_ref, xf_ref, pool_ref, wsa1_ref, bsa1_ref, wsa2_ref,
               bsa2_ref, wth1_ref, bth1_ref, wth2_ref, bth2_ref, gate_ref,
               wtr_ref, btr_ref, wfc1_ref, bfc1_ref, wfc2_ref, bfc2_ref,
               logit_ref, fused_ref):
    xp = xp_ref[...]
    xf = xf_ref[...]
    sp = jnp.concatenate([xp, xf], axis=1)
    h = jnp.maximum(jnp.dot(sp, wsa1_ref[...],
                            preferred_element_type=jnp.float32)
                    + bsa1_ref[...], 0.0)
    attn = jax.nn.sigmoid(jnp.dot(h, wsa2_ref[...],
                                  preferred_element_type=jnp.float32)
                          + bsa2_ref[...])
    fsp = sp * attn

    rows = lax.broadcasted_iota(jnp.int32, (_TOTAL, 1), 0)
    rowt = (rows // _N) % _T
    h2 = jnp.zeros((_TOTAL, _FD), jnp.float32)
    for t in range(_T):
        ht = jnp.dot(fsp, wth1_ref[t], preferred_element_type=jnp.float32)
        h2 = h2 + jnp.where(rowt == t, ht + bth1_ref[t], 0.0)
    h2 = jnp.maximum(h2, 0.0)
    a2 = jnp.zeros((_TOTAL, 1), jnp.float32)
    for t in range(_T):
        at = jnp.sum(h2 * wth2_ref[t], axis=1, keepdims=True)
        a2 = a2 + jnp.where(rowt == t, at + bth2_ref[t], 0.0)
    alpha = jax.nn.sigmoid(a2)

    pool = pool_ref[...]
    g_phy = jnp.dot(pool, alpha * xp, preferred_element_type=jnp.float32)
    g_fun = jnp.dot(pool, (1.0 - alpha) * xf, preferred_element_type=jnp.float32)
    gate = gate_ref[...]
    comb = gate * g_phy + (1.0 - gate) * g_fun
    fused = jnp.maximum(jnp.dot(comb, wtr_ref[...],
                                preferred_element_type=jnp.float32)
                        + btr_ref[...], 0.0)
    hfc = jnp.maximum(jnp.dot(fused, wfc1_ref[...],
                              preferred_element_type=jnp.float32)
                      + bfc1_ref[...], 0.0)
    lg = jnp.dot(hfc, wfc2_ref[...],
                 preferred_element_type=jnp.float32) + bfc2_ref[...]
    cols = lax.broadcasted_iota(jnp.int32, (_B, _H2), 1)
    cmask = cols < _NCLS
    mx = jnp.max(jnp.where(cmask, lg, -1e30), axis=1, keepdims=True)
    e = jnp.where(cmask, jnp.exp(lg - mx), 0.0)
    se = jnp.sum(e, axis=1, keepdims=True)
    logit_ref[...] = lg - mx - jnp.log(se)
    fused_ref[...] = fused


def _tail(xp, xf, p):
    wth1_t = jnp.transpose(p["W_th1"], (0, 2, 1))
    bth1 = p["b_th1"][:, None, :]
    wth2 = p["W_th2"]
    bth2 = p["b_th2"][:, None, :]
    gate = jnp.clip(p["gate"], 0.0, 1.0)
    wfc2_t = jnp.zeros((_H2, _H2), jnp.float32).at[:, :_NCLS].set(p["W_fc2"].T)
    bfc2 = jnp.zeros((1, _H2), jnp.float32).at[0, :_NCLS].set(p["b_fc2"])
    bidx = jnp.arange(_TOTAL, dtype=jnp.int32) // _M
    pool = (bidx[None, :] == jnp.arange(_B, dtype=jnp.int32)[:, None]
            ).astype(jnp.float32) * (1.0 / _M)
    logits_pad, fused = pl.pallas_call(
        _tail_body,
        out_shape=(jax.ShapeDtypeStruct((_B, _H2), jnp.float32),
                   jax.ShapeDtypeStruct((_B, _H2), jnp.float32)),
    )(xp, xf, pool,
      p["W_sa1"].T, p["b_sa1"][None, :], p["W_sa2"].T, p["b_sa2"][None, :],
      wth1_t, bth1, wth2, bth2, gate.reshape(1, 1),
      p["W_tr"].T, p["b_tr"][None, :], p["W_fc1"].T, p["b_fc1"][None, :],
      wfc2_t, bfc2)
    return logits_pad[:, :_NCLS], gate, fused


_NE_PAD = 573440
_EPT = _NE_PAD // 32
_EPT_A = _NE_PAD // 16
_RB = 256
_NBLK = _NP // _RB
_BLK_ELEMS = _RB * _NP


def _sc_deg_body(src_hbm, zeros_hbm, out_hbm, src_v, vals_v, idx_v, deg_sh):
    c = lax.axis_index("c")
    t = lax.axis_index("s")
    base = (t * 2 + c) * _EPT
    pltpu.sync_copy(src_hbm.at[pl.ds(base, _EPT)], src_v)

    seg = _NP // 16
    pltpu.sync_copy(zeros_hbm.at[pl.ds(t * seg, seg)],
                    deg_sh.at[pl.ds(t * seg, seg)])
    ones16 = jnp.ones((16,), jnp.float32)
    for g in range(8):
        vals_v[pl.ds(g * 16, 16)] = ones16
    plsc.subcore_barrier()

    def chunk_it(cc, carry):
        for g in range(8):
            idx_v[pl.ds(g * 16, 16)] = src_v[pl.ds(cc * 128 + g * 16, 16)]
        pltpu.sync_copy(vals_v, deg_sh.at[idx_v], add=True)
        return carry
    lax.fori_loop(0, _EPT // 128, chunk_it, 0)
    plsc.subcore_barrier()

    pltpu.sync_copy(deg_sh.at[pl.ds(t * seg, seg)],
                    out_hbm.at[pl.ds(c * _NP + t * seg, seg)])


def _sc_deg(src_pad):
    mesh = plsc.VectorSubcoreMesh(core_axis_name="c", subcore_axis_name="s")
    zeros = jnp.zeros((_NP,), jnp.float32)
    f = functools.partial(
        pl.kernel, mesh=mesh,
        out_type=jax.ShapeDtypeStruct((2 * _NP,), jnp.float32),
        scratch_types=[
            pltpu.VMEM((_EPT,), jnp.int32),
            pltpu.VMEM((128,), jnp.float32),
            pltpu.VMEM((128,), jnp.int32),
            pltpu.VMEM_SHARED((_NP,), jnp.float32),
        ],
    )(_sc_deg_body)
    return f(src_pad, zeros).reshape(2, _NP)


def _dinv_body(degp_ref, o_ref):
    deg = jnp.sum(degp_ref[...], axis=0, keepdims=True)
    o_ref[...] = jnp.where(deg > 0.0,
                           lax.rsqrt(jnp.maximum(deg, 1e-12)), 0.0)


def _dinv_tc(deg_part):
    return pl.pallas_call(
        _dinv_body,
        out_shape=jax.ShapeDtypeStruct((1, _NP), jnp.float32),
    )(deg_part)


_ECH = 8960
_DUMP = 128


def _sc_adj_body(src_hbm, dst_hbm, zeros_hbm, a_hbm,
                 src_v, dst_v, vals_v, idx_v, blk_sh):
    c = lax.axis_index("c")
    t = lax.axis_index("s")
    base = t * _EPT_A

    seg = _BLK_ELEMS // 16

    for pp in range(_NBLK // 2):
        row0 = c * (_NBLK // 2) * _RB + pp * _RB

        pltpu.sync_copy(zeros_hbm.at[pl.ds(t * seg, seg)],
                        blk_sh.at[pl.ds(t * seg, seg)])
        plsc.subcore_barrier()

        def chunk_it(cc, carry):
            e0 = base + cc * _ECH
            pltpu.sync_copy(src_hbm.at[pl.ds(e0, _ECH)], src_v)
            pltpu.sync_copy(dst_hbm.at[pl.ds(e0, _ECH)], dst_v)

            def sub_it(ss, carry2):
                for g in range(8):
                    off = ss * 128 + g * 16
                    s16 = src_v[pl.ds(off, 16)]
                    d16 = dst_v[pl.ds(off, 16)]
                    rel = d16 - row0
                    inr = (rel >= 0) & (rel < _RB)
                    flat = rel * _NP + s16
                    dump = (_BLK_ELEMS
                            + lax.broadcasted_iota(jnp.int32, (16,), 0)
                            + g * 16)
                    idx_v[pl.ds(off, 16)] = jnp.where(inr, flat, dump)
                    vals_v[pl.ds(off, 16)] = jnp.where(inr, 1.0, 0.0)
                return carry2
            lax.fori_loop(0, _ECH // 128, sub_it, 0)
            pltpu.sync_copy(vals_v, blk_sh.at[idx_v], add=True)
            return carry
        lax.fori_loop(0, _EPT_A // _ECH, chunk_it, 0)
        plsc.subcore_barrier()

        pltpu.sync_copy(blk_sh.at[pl.ds(t * seg, seg)],
                        a_hbm.at[pl.ds(row0 * _NP + t * seg, seg)])
        plsc.subcore_barrier()


def _sc_adj(src_pad, dst_pad):
    mesh = plsc.VectorSubcoreMesh(core_axis_name="c", subcore_axis_name="s")
    zeros = jnp.zeros((_BLK_ELEMS,), jnp.float32)
    f = functools.partial(
        pl.kernel, mesh=mesh,
        out_type=jax.ShapeDtypeStruct((_NP * _NP,), jnp.float32),
        scratch_types=[
            pltpu.VMEM((_ECH,), jnp.int32),
            pltpu.VMEM((_ECH,), jnp.int32),
            pltpu.VMEM((_ECH,), jnp.float32),
            pltpu.VMEM((_ECH,), jnp.int32),
            pltpu.VMEM_SHARED((_BLK_ELEMS + _DUMP,), jnp.float32),
        ],
    )(_sc_adj_body)
    return f(src_pad, dst_pad, zeros).reshape(_NP, _NP)


def _build_adj(src, dst):
    pad = jnp.full((_NE_PAD - _NE,), _NP - 1, jnp.int32)
    src_pad = jnp.concatenate([src, pad])
    dst_pad = jnp.concatenate([dst, pad])
    deg_part = _sc_deg(src_pad)
    dinv = _dinv_tc(deg_part).reshape(_NP, 1)
    cmat = _sc_adj(src_pad, dst_pad)
    return cmat, dinv


def kernel(phy_x, phy_edge_index, fun_x, params):
    p = params
    src = phy_edge_index[0]
    dst = phy_edge_index[1]
    a, dinv_col = _build_adj(src, dst)

    emb = (p["temp_embed"] + p["spat_embed"]).reshape(_M, _IN)
    emb_phy = jnp.broadcast_to(emb[None], (_B, _M, _IN)).reshape(_TOTAL, _IN)
    emb_phy = jnp.pad(emb_phy, ((0, _NP - _TOTAL), (0, 0)))
    emb_fun = jnp.pad(emb[None], ((0, 0), (0, _MP - _M), (0, 0)))
    emb_fun = jnp.broadcast_to(emb_fun, (_B, _MP, _IN)).reshape(_B * _MP, _IN)

    xp = jnp.pad(phy_x, ((0, _NP - _TOTAL), (0, 0)))
    xf3 = jnp.pad(fun_x.reshape(_B, _M, _IN), ((0, 0), (0, _MP - _M), (0, 0)))
    xf = xf3.reshape(_B * _MP, _IN)

    x = _embed_add(xp, emb_phy)
    y = _embed_add(xf, emb_fun)

    wn = _fun_adj(xf3)

    def pad_w(wmat, cin_p, cout_p):
        w2 = jnp.zeros((cin_p, cout_p), jnp.float32)
        return w2.at[:wmat.shape[0], :wmat.shape[1]].set(wmat)

    for li, lp in enumerate(p["phy_convs"]):
        cin, cout = lp["W0"].shape
        cout_p = max(cout, 128)
        w0 = pad_w(lp["W0"], cin, cout_p)
        w1 = pad_w(lp["W1"], cin, cout_p)
        b = pad_w(lp["b"][None, :], 1, cout_p)
        wres_t = pad_w(lp["Wres"].T, cin, cout_p)
        bres = pad_w(lp["bres"][None, :], 1, cout_p)
        gamma = pad_w(lp["gamma"][None, :], 1, cout_p)
        beta = pad_w(lp["beta"][None, :], 1, cout_p)
        pre = _phy_mm(a, dinv_col, x, w0, w1, b)
        x = _bnres(pre, x, wres_t, bres, gamma, beta, fun_layout=False)

    for li, lp in enumerate(p["fun_convs"]):
        cin, cout = lp["W0"].shape
        cout_p = max(cout, 128)
        w0 = pad_w(lp["W0"], cin, cout_p)
        w1 = pad_w(lp["W1"], cin, cout_p)
        b = pad_w(lp["b"][None, :], 1, cout_p)
        wres_t = pad_w(lp["Wres"].T, cin, cout_p)
        bres = pad_w(lp["bres"][None, :], 1, cout_p)
        gamma = pad_w(lp["gamma"][None, :], 1, cout_p)
        beta = pad_w(lp["beta"][None, :], 1, cout_p)
        pre = _fun_mm(wn, y, w0, w1, b)
        y = _bnres(pre, y, wres_t, bres, gamma, beta, fun_layout=True)

    x_phy = x[:_TOTAL, :_FD]
    x_fun = y.reshape(_B, _MP, -1)[:, :_M, :_FD].reshape(_TOTAL, _FD)

    return _tail(x_phy, x_fun, p)
